# baseline (speedup 1.0000x reference)
import jax
import jax.numpy as jnp
from jax import lax
from jax.experimental import pallas as pl
from jax.experimental.pallas import tpu as pltpu, tpu_sc as plsc

N, C = 16384, 1000
NC, NS, L = 2, 16, 16
NW = NC * NS
BPW = N // NW
CHUNK = 128
NCHUNK = BPW // CHUNK
SUB = CHUNK // L


def _body(prob_hbm, tgt_hbm, rwd_hbm, out_hbm, tgt_v, rwd_v, idx_v, val_v,
          acc_v, sem):
    c = lax.axis_index("c")
    s = lax.axis_index("s")
    wid = s * NC + c
    base = wid * BPW
    pltpu.sync_copy(tgt_hbm.at[pl.ds(base, BPW)], tgt_v)
    pltpu.sync_copy(rwd_hbm.at[pl.ds(base, BPW)], rwd_v)
    lane = lax.iota(jnp.int32, 16)
    acc = jnp.zeros((L,), jnp.float32)
    for j in range(NCHUNK):
        for k in range(SUB):
            off = j * CHUNK + k * L
            t = tgt_v[pl.ds(off, L)]
            idx_v[pl.ds(k * L, L)] = (base + off + lane) * C + t
        pltpu.async_copy(prob_hbm.at[idx_v], val_v, sem).wait()
        for k in range(SUB):
            off = j * CHUNK + k * L
            acc = acc + val_v[pl.ds(k * L, L)] * rwd_v[pl.ds(off, L)]
    acc_v[...] = acc * (-1.0 / N)
    pltpu.sync_copy(acc_v, out_hbm.at[wid])


@jax.jit
def _ganloss(prob_flat, target, reward):
    mesh = plsc.VectorSubcoreMesh(core_axis_name="c", subcore_axis_name="s")
    run = pl.kernel(
        _body,
        out_type=jax.ShapeDtypeStruct((NW, L), jnp.float32),
        mesh=mesh,
        scratch_types=[
            pltpu.VMEM((BPW,), jnp.int32),
            pltpu.VMEM((BPW,), jnp.float32),
            pltpu.VMEM((CHUNK,), jnp.int32),
            pltpu.VMEM((CHUNK,), jnp.float32),
            pltpu.VMEM((L,), jnp.float32),
            pltpu.SemaphoreType.DMA,
        ],
    )
    partials = run(prob_flat, target, reward)
    return jnp.sum(partials)


def kernel(prob, target, reward):
    return _ganloss(prob.reshape(-1), target.astype(jnp.int32), reward)


# speedup vs baseline: 1.8534x; 1.8534x over previous
import jax
import jax.numpy as jnp
from jax import lax
from jax.experimental import pallas as pl
from jax.experimental.pallas import tpu as pltpu

N, C = 16384, 1000
BLK = 2048
GRID = N // BLK


def _body(tgt_ref, rwd_ref, prob_ref, out_ref):
    g = pl.program_id(0)
    tgt = tgt_ref[...]
    rwd = rwd_ref[...]
    pb = prob_ref[...]
    cols = lax.broadcasted_iota(jnp.int32, (BLK, C), 1)
    picked = jnp.where(cols == tgt[:, None], pb, 0.0)
    partial = jnp.sum(jnp.sum(picked, axis=1) * rwd)

    @pl.when(g == 0)
    def _():
        out_ref[0, 0] = 0.0

    out_ref[0, 0] += partial * (-1.0 / N)


@jax.jit
def _ganloss(prob, target, reward):
    out = pl.pallas_call(
        _body,
        grid=(GRID,),
        in_specs=[
            pl.BlockSpec((BLK,), lambda g: (g,)),
            pl.BlockSpec((BLK,), lambda g: (g,)),
            pl.BlockSpec((BLK, C), lambda g: (g, 0)),
        ],
        out_specs=pl.BlockSpec(
            (1, 1), lambda g: (0, 0), memory_space=pltpu.SMEM
        ),
        out_shape=jax.ShapeDtypeStruct((1, 1), jnp.float32),
    )(target, reward, prob)
    return out[0, 0]


def kernel(prob, target, reward):
    return _ganloss(prob, target.astype(jnp.int32), reward)


# speedup vs baseline: 6.9260x; 3.7370x over previous
import jax
import jax.numpy as jnp
from jax import lax
from jax.experimental import pallas as pl
from jax.experimental.pallas import tpu as pltpu

N, C = 16384, 1000
BC = 2048
GRID = N // BC


def _body(tgt_ref, rwd_ref, pt_ref, out_ref):
    g = pl.program_id(0)
    tgt = tgt_ref[...]
    rwd = rwd_ref[...]
    pb = pt_ref[...]
    rows = lax.broadcasted_iota(jnp.int32, (C, BC), 0)
    picked = jnp.where(rows == tgt[None, :], pb, 0.0)
    partial = jnp.sum(jnp.sum(picked, axis=0) * rwd)

    @pl.when(g == 0)
    def _():
        out_ref[0, 0] = 0.0

    out_ref[0, 0] += partial * (-1.0 / N)


@jax.jit
def _ganloss(pt, target, reward):
    out = pl.pallas_call(
        _body,
        grid=(GRID,),
        in_specs=[
            pl.BlockSpec((BC,), lambda g: (g,)),
            pl.BlockSpec((BC,), lambda g: (g,)),
            pl.BlockSpec((C, BC), lambda g: (0, g)),
        ],
        out_specs=pl.BlockSpec(
            (1, 1), lambda g: (0, 0), memory_space=pltpu.SMEM
        ),
        out_shape=jax.ShapeDtypeStruct((1, 1), jnp.float32),
    )(target, reward, pt)
    return out[0, 0]


def kernel(prob, target, reward):
    return _ganloss(prob.T, target.astype(jnp.int32), reward)
